# baseline (speedup 1.0000x reference)
import functools

import jax
import jax.numpy as jnp
from jax import lax
from jax.experimental import pallas as pl
from jax.experimental.pallas import tpu as pltpu
from jax.experimental.pallas import tpu_sc as plsc

B = 16384
D = 64
CH = 256
NSEM = 4


@functools.cache
def _build_kernel():
    info = plsc.get_sparse_core_info()
    nw = info.num_cores * info.num_subcores
    b_per_w = B // nw
    n_ch = b_per_w // CH
    mesh = plsc.VectorSubcoreMesh(core_axis_name="c", subcore_axis_name="s")

    @functools.partial(
        pl.kernel,
        mesh=mesh,
        out_type=(
            jax.ShapeDtypeStruct((B, D), jnp.float32),
            jax.ShapeDtypeStruct((B, D), jnp.float32),
        ),
        scratch_types=[
            pltpu.VMEM((CH,), jnp.int32),
            pltpu.VMEM((CH, D), jnp.float32),
            pltpu.VMEM((CH, D), jnp.float32),
            [pltpu.SemaphoreType.DMA] * NSEM,
            [pltpu.SemaphoreType.DMA] * NSEM,
            pltpu.SemaphoreType.DMA,
            pltpu.SemaphoreType.DMA,
        ],
    )
    def k(ids_hbm, ws_hbm, wa_hbm, out_s_hbm, out_a_hbm,
          idx_v, rows_s, rows_a, sems_s, sems_a, sem_os, sem_oa):
        wid = lax.axis_index("s") * info.num_cores + lax.axis_index("c")
        base = wid * b_per_w

        for ch in range(n_ch):
            pltpu.sync_copy(ids_hbm.at[pl.ds(base + ch * CH, CH)], idx_v)

            def fire(g, _):
                v = idx_v[pl.ds(g * 16, 16)]
                for l in range(16):
                    r = v[l]
                    i = g * 16 + l
                    pltpu.async_copy(
                        ws_hbm.at[pl.ds(r, 1)], rows_s.at[pl.ds(i, 1)],
                        sems_s[l % NSEM])
                    pltpu.async_copy(
                        wa_hbm.at[pl.ds(r, 1)], rows_a.at[pl.ds(i, 1)],
                        sems_a[l % NSEM])
                return 0

            lax.fori_loop(0, CH // 16, fire, 0)
            for q in range(NSEM):
                pltpu.make_async_copy(
                    ws_hbm.at[pl.ds(0, CH // NSEM)],
                    rows_s.at[pl.ds(0, CH // NSEM)], sems_s[q]).wait()
                pltpu.make_async_copy(
                    wa_hbm.at[pl.ds(0, CH // NSEM)],
                    rows_a.at[pl.ds(0, CH // NSEM)], sems_a[q]).wait()
            pltpu.async_copy(
                rows_s, out_s_hbm.at[pl.ds(base + ch * CH, CH)], sem_os).wait()
            pltpu.async_copy(
                rows_a, out_a_hbm.at[pl.ds(base + ch * CH, CH)], sem_oa).wait()

    return k


def kernel(instance_ids, W_shape, W_appearance):
    ids = instance_ids.astype(jnp.int32)
    return _build_kernel()(ids, W_shape, W_appearance)


# speedup vs baseline: 1.0014x; 1.0014x over previous
import functools

import jax
import jax.numpy as jnp
from jax import lax
from jax.experimental import pallas as pl
from jax.experimental.pallas import tpu as pltpu
from jax.experimental.pallas import tpu_sc as plsc

B = 16384
D = 64
CH = 256


@functools.cache
def _build_kernel():
    info = plsc.get_sparse_core_info()
    nw = info.num_cores * info.num_subcores
    b_per_w = B // nw
    n_ch = b_per_w // CH
    mesh = plsc.VectorSubcoreMesh(core_axis_name="c", subcore_axis_name="s")

    @functools.partial(
        pl.kernel,
        mesh=mesh,
        out_type=(
            jax.ShapeDtypeStruct((B, D), jnp.float32),
            jax.ShapeDtypeStruct((B, D), jnp.float32),
        ),
        scratch_types=[
            pltpu.VMEM((CH,), jnp.int32),
            pltpu.VMEM((CH, D), jnp.float32),
            pltpu.VMEM((CH, D), jnp.float32),
            pltpu.SemaphoreType.DMA,
            pltpu.SemaphoreType.DMA,
            pltpu.SemaphoreType.DMA,
        ],
        compiler_params=pltpu.CompilerParams(needs_layout_passes=False),
    )
    def k(ids_hbm, ws_hbm, wa_hbm, out_s_hbm, out_a_hbm,
          idx_v, rows_s, rows_a, sem_g, sem_os, sem_oa):
        wid = lax.axis_index("s") * info.num_cores + lax.axis_index("c")
        base = wid * b_per_w

        for ch in range(n_ch):
            pltpu.sync_copy(ids_hbm.at[pl.ds(base + ch * CH, CH)], idx_v)

            def fire(g):
                v = idx_v[pl.ds(g * 16, 16)]
                for l in range(16):
                    r = v[l]
                    i = g * 16 + l
                    pltpu.async_copy(
                        ws_hbm.at[pl.ds(r, 1)], rows_s.at[pl.ds(i, 1)], sem_g)
                    pltpu.async_copy(
                        wa_hbm.at[pl.ds(r, 1)], rows_a.at[pl.ds(i, 1)], sem_g)

            plsc.parallel_loop(0, CH // 16, 1, unroll=4)(fire)
            pltpu.make_async_copy(ws_hbm.at[pl.ds(0, CH)], rows_s, sem_g).wait()
            pltpu.make_async_copy(wa_hbm.at[pl.ds(0, CH)], rows_a, sem_g).wait()
            pltpu.async_copy(
                rows_s, out_s_hbm.at[pl.ds(base + ch * CH, CH)], sem_os).wait()
            pltpu.async_copy(
                rows_a, out_a_hbm.at[pl.ds(base + ch * CH, CH)], sem_oa).wait()

    return k


def kernel(instance_ids, W_shape, W_appearance):
    ids = instance_ids.astype(jnp.int32)
    return _build_kernel()(ids, W_shape, W_appearance)
